# baseline (speedup 1.0000x reference)
import functools

import jax
import jax.numpy as jnp
import numpy as np
from jax import lax
from jax.experimental import pallas as pl
from jax.experimental.pallas import tpu as pltpu
from jax.experimental.pallas import tpu_sc as plsc

_MASKRATIO = 0.75
_PH = 16
_PW = 16
_C = 3
_H = 4096
_W = 4096
_NPH = _H // _PH
_NPW = _W // _PW

_L = 16
_NCHUNK = _C * _H * _W // _L
_S = 2048
_NSEG = _NCHUNK // _S
_NC = 2
_NS = 16
_NW = _NC * _NS
_SEG_PER_W = _NSEG // _NW


def _tf2x32(k1, k2, x1, x2):
    R0 = (13, 15, 26, 6)
    R1 = (17, 29, 16, 24)
    k1 = np.uint32(k1)
    k2 = np.uint32(k2)
    ks = (k1, k2, np.uint32(k1 ^ k2 ^ np.uint32(0x1BD11BDA)))
    x = [(x1 + ks[0]).astype(np.uint32), (x2 + ks[1]).astype(np.uint32)]
    for i in range(5):
        for r in R0 if i % 2 == 0 else R1:
            x[0] = (x[0] + x[1]).astype(np.uint32)
            x[1] = ((x[1] << np.uint32(r)) | (x[1] >> np.uint32(32 - r))).astype(
                np.uint32
            )
            x[1] = x[0] ^ x[1]
        x[0] = (x[0] + ks[(i + 1) % 3]).astype(np.uint32)
        x[1] = (x[1] + ks[(i + 2) % 3] + np.uint32(i + 1)).astype(np.uint32)
    return x[0], x[1]


def _random_bits(k1, k2, n):
    lo = np.arange(n, dtype=np.uint32)
    hi = np.zeros(n, dtype=np.uint32)
    b1, b2 = _tf2x32(k1, k2, hi, lo)
    return b1 ^ b2


def _split2(k1, k2):
    b1, b2 = _tf2x32(k1, k2, np.zeros(2, np.uint32), np.arange(2, dtype=np.uint32))
    return (b1[0], b2[0]), (b1[1], b2[1])


def _erfinv_f32(x):
    x = x.astype(np.float32)
    w = (-np.log1p((-x * x).astype(np.float32))).astype(np.float32)
    lt = w < np.float32(5.0)
    wa = (w - np.float32(2.5)).astype(np.float32)
    pa = np.float32(2.81022636e-08)
    for c in (3.43273939e-07, -3.5233877e-06, -4.39150654e-06, 0.00021858087,
              -0.00125372503, -0.00417768164, 0.246640727, 1.50140941):
        pa = (np.float32(c) + pa * wa).astype(np.float32)
    wb = (np.sqrt(np.maximum(w, np.float32(5.0)), dtype=np.float32)
          - np.float32(3.0)).astype(np.float32)
    pb = np.float32(-0.000200214257)
    for c in (0.000100950558, 0.00134934322, -0.00367342844, 0.00573950773,
              -0.0076224613, 0.00943887047, 1.00167406, 2.83297682):
        pb = (np.float32(c) + pb * wb).astype(np.float32)
    return (np.where(lt, pa, pb) * x).astype(np.float32)


def _normal_f32(k1, k2, n):
    bits = _random_bits(k1, k2, n)
    fl = ((bits >> np.uint32(9)) | np.uint32(0x3F800000)).view(np.float32)
    fl = (fl - np.float32(1.0)).astype(np.float32)
    lo = np.nextafter(np.float32(-1.0), np.float32(0.0))
    hi = np.float32(1.0)
    u = np.maximum(lo, (fl * (hi - lo) + lo).astype(np.float32))
    return (np.float32(np.sqrt(2.0)) * _erfinv_f32(u)).astype(np.float32)


@functools.cache
def _consts():
    (m1, m2), (n1, n2) = _split2(np.uint32(0), np.uint32(42))
    (a1, a2), (b1, b2) = _split2(m1, m2)
    higher = _random_bits(a1, a2, _NPH * _NPW)
    lower = _random_bits(b1, b2, _NPH * _NPW)
    span = np.uint32(10)
    mult = np.uint32(((2 ** 16) % 10) ** 2 % 10)
    draws = 1 + (((higher % span) * mult + (lower % span)) % span).astype(np.int32)
    mask = draws < (10.0 * _MASKRATIO)
    noise = _normal_f32(n1, n2, _NPH * _NPW * _PH * _PW * _C).reshape(
        _NPH * _NPW, _PH * _PW * _C
    )
    nz = (
        noise.reshape(_NPH, _NPW, _PH, _PW, _C)
        .transpose(4, 0, 2, 1, 3)
        .reshape(-1)
    )
    kk = np.arange(_NCHUNK, dtype=np.int64)
    flat = kk * _L
    y = (flat // _W) % _H
    x = flat % _W
    cmask = mask[(y // _PH) * _NPW + (x // _PW)]
    srcrel = ((kk % _S) * _L + np.where(cmask, _S * _L, 0)).astype(np.int32)
    return nz, srcrel


def _sc_body(img_hbm, nz_hbm, src_hbm, out_hbm, buf, obuf, sbuf):
    wid = lax.axis_index("s") * _NC + lax.axis_index("c")
    iota = lax.iota(jnp.int32, _L)

    def seg(s, carry):
        g = wid * _SEG_PER_W + s
        base = g * (_S * _L)
        pltpu.sync_copy(img_hbm.at[pl.ds(base, _S * _L)],
                        buf.at[pl.ds(0, _S * _L)])
        pltpu.sync_copy(nz_hbm.at[pl.ds(base, _S * _L)],
                        buf.at[pl.ds(_S * _L, _S * _L)])
        pltpu.sync_copy(src_hbm.at[pl.ds(g * _S, _S)], sbuf)

        def grp(t, c2):
            srcv = sbuf[pl.ds(t * _L, _L)]
            obase = t * (_L * _L)
            for l in range(_L):
                tv = plsc.load_gather(buf, [srcv + l])
                plsc.store_scatter(obuf, [obase + l + iota * _L], tv)
            return c2

        lax.fori_loop(0, _S // _L, grp, 0, unroll=False)
        pltpu.sync_copy(obuf, out_hbm.at[pl.ds(base, _S * _L)])
        return carry

    lax.fori_loop(0, _SEG_PER_W, seg, 0, unroll=False)


@jax.jit
def _run(img, nz, srcrel):
    mesh = plsc.VectorSubcoreMesh(
        core_axis_name="c", subcore_axis_name="s",
        num_cores=_NC, num_subcores=_NS,
    )
    f = pl.kernel(
        _sc_body,
        out_type=jax.ShapeDtypeStruct((_NCHUNK * _L,), jnp.float32),
        mesh=mesh,
        scratch_types=[
            pltpu.VMEM((2 * _S * _L,), jnp.float32),
            pltpu.VMEM((_S * _L,), jnp.float32),
            pltpu.VMEM((_S,), jnp.int32),
        ],
        compiler_params=pltpu.CompilerParams(needs_layout_passes=False),
    )
    out = f(img.reshape(-1), nz, srcrel)
    return out.reshape(_C, _H, _W)


_NZ, _SRCREL = _consts()


def kernel(img):
    return _run(img, jnp.asarray(_NZ), jnp.asarray(_SRCREL))


# speedup vs baseline: 2.0953x; 2.0953x over previous
import functools

import jax
import jax.numpy as jnp
import numpy as np
from jax import lax
from jax.experimental import pallas as pl
from jax.experimental.pallas import tpu as pltpu
from jax.experimental.pallas import tpu_sc as plsc

_MASKRATIO = 0.75
_PH = 16
_PW = 16
_C = 3
_H = 4096
_W = 4096
_NPH = _H // _PH
_NPW = _W // _PW

_L = 16
_NCHUNK = _C * _H * _W // _L
_S = 1024
_NSEG = _NCHUNK // _S
_NC = 2
_NS = 16
_NW = _NC * _NS
_SEG_PER_W = _NSEG // _NW
_NPAIR = _SEG_PER_W // 2


def _tf2x32(k1, k2, x1, x2):
    R0 = (13, 15, 26, 6)
    R1 = (17, 29, 16, 24)
    k1 = np.uint32(k1)
    k2 = np.uint32(k2)
    ks = (k1, k2, np.uint32(k1 ^ k2 ^ np.uint32(0x1BD11BDA)))
    x = [(x1 + ks[0]).astype(np.uint32), (x2 + ks[1]).astype(np.uint32)]
    for i in range(5):
        for r in R0 if i % 2 == 0 else R1:
            x[0] = (x[0] + x[1]).astype(np.uint32)
            x[1] = ((x[1] << np.uint32(r)) | (x[1] >> np.uint32(32 - r))).astype(
                np.uint32
            )
            x[1] = x[0] ^ x[1]
        x[0] = (x[0] + ks[(i + 1) % 3]).astype(np.uint32)
        x[1] = (x[1] + ks[(i + 2) % 3] + np.uint32(i + 1)).astype(np.uint32)
    return x[0], x[1]


def _random_bits(k1, k2, n):
    lo = np.arange(n, dtype=np.uint32)
    hi = np.zeros(n, dtype=np.uint32)
    b1, b2 = _tf2x32(k1, k2, hi, lo)
    return b1 ^ b2


def _split2(k1, k2):
    b1, b2 = _tf2x32(k1, k2, np.zeros(2, np.uint32), np.arange(2, dtype=np.uint32))
    return (b1[0], b2[0]), (b1[1], b2[1])


def _erfinv_f32(x):
    x = x.astype(np.float32)
    w = (-np.log1p((-x * x).astype(np.float32))).astype(np.float32)
    lt = w < np.float32(5.0)
    wa = (w - np.float32(2.5)).astype(np.float32)
    pa = np.float32(2.81022636e-08)
    for c in (3.43273939e-07, -3.5233877e-06, -4.39150654e-06, 0.00021858087,
              -0.00125372503, -0.00417768164, 0.246640727, 1.50140941):
        pa = (np.float32(c) + pa * wa).astype(np.float32)
    wb = (np.sqrt(np.maximum(w, np.float32(5.0)), dtype=np.float32)
          - np.float32(3.0)).astype(np.float32)
    pb = np.float32(-0.000200214257)
    for c in (0.000100950558, 0.00134934322, -0.00367342844, 0.00573950773,
              -0.0076224613, 0.00943887047, 1.00167406, 2.83297682):
        pb = (np.float32(c) + pb * wb).astype(np.float32)
    return (np.where(lt, pa, pb) * x).astype(np.float32)


def _normal_f32(k1, k2, n):
    bits = _random_bits(k1, k2, n)
    fl = ((bits >> np.uint32(9)) | np.uint32(0x3F800000)).view(np.float32)
    fl = (fl - np.float32(1.0)).astype(np.float32)
    lo = np.nextafter(np.float32(-1.0), np.float32(0.0))
    hi = np.float32(1.0)
    u = np.maximum(lo, (fl * (hi - lo) + lo).astype(np.float32))
    return (np.float32(np.sqrt(2.0)) * _erfinv_f32(u)).astype(np.float32)


@functools.cache
def _consts():
    (m1, m2), (n1, n2) = _split2(np.uint32(0), np.uint32(42))
    (a1, a2), (b1, b2) = _split2(m1, m2)
    higher = _random_bits(a1, a2, _NPH * _NPW)
    lower = _random_bits(b1, b2, _NPH * _NPW)
    span = np.uint32(10)
    mult = np.uint32(((2 ** 16) % 10) ** 2 % 10)
    draws = 1 + (((higher % span) * mult + (lower % span)) % span).astype(np.int32)
    mask = draws < (10.0 * _MASKRATIO)
    noise = _normal_f32(n1, n2, _NPH * _NPW * _PH * _PW * _C).reshape(
        _NPH * _NPW, _PH * _PW * _C
    )
    nz = (
        noise.reshape(_NPH, _NPW, _PH, _PW, _C)
        .transpose(4, 0, 2, 1, 3)
        .reshape(-1)
    )
    kk = np.arange(_NCHUNK, dtype=np.int64)
    flat = kk * _L
    y = (flat // _W) % _H
    x = flat % _W
    cmask = mask[(y // _PH) * _NPW + (x // _PW)]
    srcrel = ((kk % _S) * _L + np.where(cmask, _S * _L, 0)).astype(np.int32)
    return nz, srcrel


def _sc_body(img_hbm, nz_hbm, src_hbm, out_hbm,
             buf0, buf1, obuf0, obuf1, sbuf0, sbuf1,
             lsem0, lsem1, ssem0, ssem1):
    wid = lax.axis_index("s") * _NC + lax.axis_index("c")
    iota16 = lax.iota(jnp.int32, _L)
    bufs = (buf0, buf1)
    obufs = (obuf0, obuf1)
    sbufs = (sbuf0, sbuf1)
    lsems = (lsem0, lsem1)
    ssems = (ssem0, ssem1)
    seg0 = wid * _SEG_PER_W

    def _load_descs(p, s):
        base = (seg0 + s) * (_S * _L)
        sbase = (seg0 + s) * _S
        return (
            (img_hbm.at[pl.ds(base, _S * _L)], bufs[p].at[pl.ds(0, _S * _L)]),
            (nz_hbm.at[pl.ds(base, _S * _L)],
             bufs[p].at[pl.ds(_S * _L, _S * _L)]),
            (src_hbm.at[pl.ds(sbase, _S)], sbufs[p]),
        )

    def start_loads(p, s):
        for src, dst in _load_descs(p, s):
            pltpu.async_copy(src, dst, lsems[p])

    def wait_loads(p, s):
        for src, dst in _load_descs(p, s):
            pltpu.make_async_copy(src, dst, lsems[p]).wait()

    def _store_desc(p, s):
        base = (seg0 + s) * (_S * _L)
        return obufs[p], out_hbm.at[pl.ds(base, _S * _L)]

    def start_store(p, s):
        src, dst = _store_desc(p, s)
        pltpu.async_copy(src, dst, ssems[p])

    def wait_store(p, s):
        src, dst = _store_desc(p, s)
        pltpu.make_async_copy(src, dst, ssems[p]).wait()

    def merge(p):
        @plsc.parallel_loop(0, _S // _L)
        def grp(t):
            srcv = sbufs[p][pl.ds(t * _L, _L)]
            obase = t * (_L * _L)
            for l in range(_L):
                tv = plsc.load_gather(bufs[p], [srcv + l])
                plsc.store_scatter(obufs[p], [obase + l + iota16 * _L], tv)

    start_loads(0, 0)
    start_loads(1, 1)

    def pair(it, carry):
        for p in (0, 1):
            s = it * 2 + p
            wait_loads(p, s)

            @pl.when(it > 0)
            def _():
                wait_store(p, s - 2)

            merge(p)
            start_store(p, s)

            @pl.when(it < _NPAIR - 1)
            def _():
                start_loads(p, s + 2)

        return carry

    lax.fori_loop(0, _NPAIR, pair, 0, unroll=False)
    wait_store(0, _SEG_PER_W - 2)
    wait_store(1, _SEG_PER_W - 1)


@jax.jit
def _run(img, nz, srcrel):
    mesh = plsc.VectorSubcoreMesh(
        core_axis_name="c", subcore_axis_name="s",
        num_cores=_NC, num_subcores=_NS,
    )
    f = pl.kernel(
        _sc_body,
        out_type=jax.ShapeDtypeStruct((_NCHUNK * _L,), jnp.float32),
        mesh=mesh,
        scratch_types=[
            pltpu.VMEM((2 * _S * _L,), jnp.float32),
            pltpu.VMEM((2 * _S * _L,), jnp.float32),
            pltpu.VMEM((_S * _L,), jnp.float32),
            pltpu.VMEM((_S * _L,), jnp.float32),
            pltpu.VMEM((_S,), jnp.int32),
            pltpu.VMEM((_S,), jnp.int32),
            pltpu.SemaphoreType.DMA,
            pltpu.SemaphoreType.DMA,
            pltpu.SemaphoreType.DMA,
            pltpu.SemaphoreType.DMA,
        ],
        compiler_params=pltpu.CompilerParams(needs_layout_passes=False),
    )
    out = f(img.reshape(-1), nz, srcrel)
    return out.reshape(_C, _H, _W)


_NZ, _SRCREL = _consts()


def kernel(img):
    return _run(img, jnp.asarray(_NZ), jnp.asarray(_SRCREL))


# speedup vs baseline: 2.1256x; 1.0145x over previous
import functools

import jax
import jax.numpy as jnp
import numpy as np
from jax import lax
from jax.experimental import pallas as pl
from jax.experimental.pallas import tpu as pltpu
from jax.experimental.pallas import tpu_sc as plsc

_MASKRATIO = 0.75
_PH = 16
_PW = 16
_C = 3
_H = 4096
_W = 4096
_NPH = _H // _PH
_NPW = _W // _PW

_L = 16
_NCHUNK = _C * _H * _W // _L
_S = 512
_NSEG = _NCHUNK // _S
_NC = 2
_NS = 16
_NW = _NC * _NS
_SEG_PER_W = _NSEG // _NW
_NBUF = 4
_NROUND = _SEG_PER_W // _NBUF


def _tf2x32(k1, k2, x1, x2):
    R0 = (13, 15, 26, 6)
    R1 = (17, 29, 16, 24)
    k1 = np.uint32(k1)
    k2 = np.uint32(k2)
    ks = (k1, k2, np.uint32(k1 ^ k2 ^ np.uint32(0x1BD11BDA)))
    x = [(x1 + ks[0]).astype(np.uint32), (x2 + ks[1]).astype(np.uint32)]
    for i in range(5):
        for r in R0 if i % 2 == 0 else R1:
            x[0] = (x[0] + x[1]).astype(np.uint32)
            x[1] = ((x[1] << np.uint32(r)) | (x[1] >> np.uint32(32 - r))).astype(
                np.uint32
            )
            x[1] = x[0] ^ x[1]
        x[0] = (x[0] + ks[(i + 1) % 3]).astype(np.uint32)
        x[1] = (x[1] + ks[(i + 2) % 3] + np.uint32(i + 1)).astype(np.uint32)
    return x[0], x[1]


def _random_bits(k1, k2, n):
    lo = np.arange(n, dtype=np.uint32)
    hi = np.zeros(n, dtype=np.uint32)
    b1, b2 = _tf2x32(k1, k2, hi, lo)
    return b1 ^ b2


def _split2(k1, k2):
    b1, b2 = _tf2x32(k1, k2, np.zeros(2, np.uint32), np.arange(2, dtype=np.uint32))
    return (b1[0], b2[0]), (b1[1], b2[1])


def _erfinv_f32(x):
    x = x.astype(np.float32)
    w = (-np.log1p((-x * x).astype(np.float32))).astype(np.float32)
    lt = w < np.float32(5.0)
    wa = (w - np.float32(2.5)).astype(np.float32)
    pa = np.float32(2.81022636e-08)
    for c in (3.43273939e-07, -3.5233877e-06, -4.39150654e-06, 0.00021858087,
              -0.00125372503, -0.00417768164, 0.246640727, 1.50140941):
        pa = (np.float32(c) + pa * wa).astype(np.float32)
    wb = (np.sqrt(np.maximum(w, np.float32(5.0)), dtype=np.float32)
          - np.float32(3.0)).astype(np.float32)
    pb = np.float32(-0.000200214257)
    for c in (0.000100950558, 0.00134934322, -0.00367342844, 0.00573950773,
              -0.0076224613, 0.00943887047, 1.00167406, 2.83297682):
        pb = (np.float32(c) + pb * wb).astype(np.float32)
    return (np.where(lt, pa, pb) * x).astype(np.float32)


def _normal_f32(k1, k2, n):
    bits = _random_bits(k1, k2, n)
    fl = ((bits >> np.uint32(9)) | np.uint32(0x3F800000)).view(np.float32)
    fl = (fl - np.float32(1.0)).astype(np.float32)
    lo = np.nextafter(np.float32(-1.0), np.float32(0.0))
    hi = np.float32(1.0)
    u = np.maximum(lo, (fl * (hi - lo) + lo).astype(np.float32))
    return (np.float32(np.sqrt(2.0)) * _erfinv_f32(u)).astype(np.float32)


@functools.cache
def _consts():
    (m1, m2), (n1, n2) = _split2(np.uint32(0), np.uint32(42))
    (a1, a2), (b1, b2) = _split2(m1, m2)
    higher = _random_bits(a1, a2, _NPH * _NPW)
    lower = _random_bits(b1, b2, _NPH * _NPW)
    span = np.uint32(10)
    mult = np.uint32(((2 ** 16) % 10) ** 2 % 10)
    draws = 1 + (((higher % span) * mult + (lower % span)) % span).astype(np.int32)
    mask = draws < (10.0 * _MASKRATIO)
    noise = _normal_f32(n1, n2, _NPH * _NPW * _PH * _PW * _C).reshape(
        _NPH * _NPW, _PH * _PW * _C
    )
    nz = (
        noise.reshape(_NPH, _NPW, _PH, _PW, _C)
        .transpose(4, 0, 2, 1, 3)
        .reshape(-1)
    )
    kk = np.arange(_NCHUNK, dtype=np.int64)
    flat = kk * _L
    y = (flat // _W) % _H
    x = flat % _W
    cmask = mask[(y // _PH) * _NPW + (x // _PW)]
    srcrel = ((kk % _S) * _L + np.where(cmask, _S * _L, 0)).astype(np.int32)
    return nz, srcrel


def _sc_body(img_hbm, nz_hbm, src_hbm, out_hbm, *scratch):
    wid = lax.axis_index("s") * _NC + lax.axis_index("c")
    iota16 = lax.iota(jnp.int32, _L)
    bufs = scratch[0:_NBUF]
    obufs = scratch[_NBUF:2 * _NBUF]
    sbufs = scratch[2 * _NBUF:3 * _NBUF]
    lsems = scratch[3 * _NBUF:4 * _NBUF]
    ssems = scratch[4 * _NBUF:5 * _NBUF]
    seg0 = wid * _SEG_PER_W

    def _load_descs(p, s):
        base = (seg0 + s) * (_S * _L)
        sbase = (seg0 + s) * _S
        return (
            (img_hbm.at[pl.ds(base, _S * _L)], bufs[p].at[pl.ds(0, _S * _L)]),
            (nz_hbm.at[pl.ds(base, _S * _L)],
             bufs[p].at[pl.ds(_S * _L, _S * _L)]),
            (src_hbm.at[pl.ds(sbase, _S)], sbufs[p]),
        )

    def start_loads(p, s):
        for src, dst in _load_descs(p, s):
            pltpu.async_copy(src, dst, lsems[p])

    def wait_loads(p, s):
        for src, dst in _load_descs(p, s):
            pltpu.make_async_copy(src, dst, lsems[p]).wait()

    def _store_desc(p, s):
        base = (seg0 + s) * (_S * _L)
        return obufs[p], out_hbm.at[pl.ds(base, _S * _L)]

    def start_store(p, s):
        src, dst = _store_desc(p, s)
        pltpu.async_copy(src, dst, ssems[p])

    def wait_store(p, s):
        src, dst = _store_desc(p, s)
        pltpu.make_async_copy(src, dst, ssems[p]).wait()

    def merge(p):
        @plsc.parallel_loop(0, _S // _L)
        def grp(t):
            srcv = sbufs[p][pl.ds(t * _L, _L)]
            obase = t * (_L * _L)
            for l in range(_L):
                tv = plsc.load_gather(bufs[p], [srcv + l])
                plsc.store_scatter(obufs[p], [obase + l + iota16 * _L], tv)

    for p in range(_NBUF):
        start_loads(p, p)

    def rnd(it, carry):
        for p in range(_NBUF):
            s = it * _NBUF + p
            wait_loads(p, s)

            @pl.when(it > 0)
            def _():
                wait_store(p, s - _NBUF)

            merge(p)
            start_store(p, s)

            @pl.when(it < _NROUND - 1)
            def _():
                start_loads(p, s + _NBUF)

        return carry

    lax.fori_loop(0, _NROUND, rnd, 0, unroll=False)
    for p in range(_NBUF):
        wait_store(p, _SEG_PER_W - _NBUF + p)


@jax.jit
def _run(img, nz, srcrel):
    mesh = plsc.VectorSubcoreMesh(
        core_axis_name="c", subcore_axis_name="s",
        num_cores=_NC, num_subcores=_NS,
    )
    f = pl.kernel(
        _sc_body,
        out_type=jax.ShapeDtypeStruct((_NCHUNK * _L,), jnp.float32),
        mesh=mesh,
        scratch_types=(
            [pltpu.VMEM((2 * _S * _L,), jnp.float32)] * _NBUF
            + [pltpu.VMEM((_S * _L,), jnp.float32)] * _NBUF
            + [pltpu.VMEM((_S,), jnp.int32)] * _NBUF
            + [pltpu.SemaphoreType.DMA] * (2 * _NBUF)
        ),
        compiler_params=pltpu.CompilerParams(needs_layout_passes=False),
    )
    out = f(img.reshape(-1), nz, srcrel)
    return out.reshape(_C, _H, _W)


_NZ, _SRCREL = _consts()


def kernel(img):
    return _run(img, jnp.asarray(_NZ), jnp.asarray(_SRCREL))


# speedup vs baseline: 3.8779x; 1.8244x over previous
import functools

import jax
import jax.numpy as jnp
import numpy as np
from jax import lax
from jax.experimental import pallas as pl
from jax.experimental.pallas import tpu as pltpu
from jax.experimental.pallas import tpu_sc as plsc

_MASKRATIO = 0.75
_PH = 16
_PW = 16
_C = 3
_H = 4096
_W = 4096
_NPH = _H // _PH
_NPW = _W // _PW

_L = 16
_NCHUNK = _C * _H * _W // _L
_ROWS = _C * _H
_SR = 8
_SW = 2048
_S = _SR * _SW // _L
_XSPLIT = _W // _SW
_NSEG = _NCHUNK // _S
_NC = 2
_NS = 16
_NW = _NC * _NS
_SEG_PER_W = _NSEG // _NW
_NBUF = 2
_NROUND = _SEG_PER_W // _NBUF


def _tf2x32(k1, k2, x1, x2):
    R0 = (13, 15, 26, 6)
    R1 = (17, 29, 16, 24)
    k1 = np.uint32(k1)
    k2 = np.uint32(k2)
    ks = (k1, k2, np.uint32(k1 ^ k2 ^ np.uint32(0x1BD11BDA)))
    x = [(x1 + ks[0]).astype(np.uint32), (x2 + ks[1]).astype(np.uint32)]
    for i in range(5):
        for r in R0 if i % 2 == 0 else R1:
            x[0] = (x[0] + x[1]).astype(np.uint32)
            x[1] = ((x[1] << np.uint32(r)) | (x[1] >> np.uint32(32 - r))).astype(
                np.uint32
            )
            x[1] = x[0] ^ x[1]
        x[0] = (x[0] + ks[(i + 1) % 3]).astype(np.uint32)
        x[1] = (x[1] + ks[(i + 2) % 3] + np.uint32(i + 1)).astype(np.uint32)
    return x[0], x[1]


def _random_bits(k1, k2, n):
    lo = np.arange(n, dtype=np.uint32)
    hi = np.zeros(n, dtype=np.uint32)
    b1, b2 = _tf2x32(k1, k2, hi, lo)
    return b1 ^ b2


def _split2(k1, k2):
    b1, b2 = _tf2x32(k1, k2, np.zeros(2, np.uint32), np.arange(2, dtype=np.uint32))
    return (b1[0], b2[0]), (b1[1], b2[1])


def _erfinv_f32(x):
    x = x.astype(np.float32)
    w = (-np.log1p((-x * x).astype(np.float32))).astype(np.float32)
    lt = w < np.float32(5.0)
    wa = (w - np.float32(2.5)).astype(np.float32)
    pa = np.float32(2.81022636e-08)
    for c in (3.43273939e-07, -3.5233877e-06, -4.39150654e-06, 0.00021858087,
              -0.00125372503, -0.00417768164, 0.246640727, 1.50140941):
        pa = (np.float32(c) + pa * wa).astype(np.float32)
    wb = (np.sqrt(np.maximum(w, np.float32(5.0)), dtype=np.float32)
          - np.float32(3.0)).astype(np.float32)
    pb = np.float32(-0.000200214257)
    for c in (0.000100950558, 0.00134934322, -0.00367342844, 0.00573950773,
              -0.0076224613, 0.00943887047, 1.00167406, 2.83297682):
        pb = (np.float32(c) + pb * wb).astype(np.float32)
    return (np.where(lt, pa, pb) * x).astype(np.float32)


def _normal_f32(k1, k2, n):
    bits = _random_bits(k1, k2, n)
    fl = ((bits >> np.uint32(9)) | np.uint32(0x3F800000)).view(np.float32)
    fl = (fl - np.float32(1.0)).astype(np.float32)
    lo = np.nextafter(np.float32(-1.0), np.float32(0.0))
    hi = np.float32(1.0)
    u = np.maximum(lo, (fl * (hi - lo) + lo).astype(np.float32))
    return (np.float32(np.sqrt(2.0)) * _erfinv_f32(u)).astype(np.float32)


@functools.cache
def _consts():
    (m1, m2), (n1, n2) = _split2(np.uint32(0), np.uint32(42))
    (a1, a2), (b1, b2) = _split2(m1, m2)
    higher = _random_bits(a1, a2, _NPH * _NPW)
    lower = _random_bits(b1, b2, _NPH * _NPW)
    span = np.uint32(10)
    mult = np.uint32(((2 ** 16) % 10) ** 2 % 10)
    draws = 1 + (((higher % span) * mult + (lower % span)) % span).astype(np.int32)
    mask = draws < (10.0 * _MASKRATIO)
    noise = _normal_f32(n1, n2, _NPH * _NPW * _PH * _PW * _C).reshape(
        _NPH * _NPW, _PH * _PW * _C
    )
    nz = (
        noise.reshape(_NPH, _NPW, _PH, _PW, _C)
        .transpose(4, 0, 2, 1, 3)
        .reshape(_ROWS, _W)
    )
    g = np.arange(_NSEG, dtype=np.int64)[:, None]
    c = np.arange(_S, dtype=np.int64)[None, :]
    row = (g // _XSPLIT) * _SR + c // (_SW // _L)
    col = (g % _XSPLIT) * _SW + (c % (_SW // _L)) * _L
    y = row % _H
    msel = mask[(y // _PH) * _NPW + (col // _PW)].astype(np.int32)
    return nz, msel.reshape(-1)


def _sc_body(img_hbm, nz_hbm, src_hbm, out_hbm, *scratch):
    wid = lax.axis_index("s") * _NC + lax.axis_index("c")
    iota16 = lax.iota(jnp.int32, _L)
    bufs = scratch[0:_NBUF]
    obufs = scratch[_NBUF:2 * _NBUF]
    sbufs = scratch[2 * _NBUF:3 * _NBUF]
    lsems = scratch[3 * _NBUF:4 * _NBUF]
    ssems = scratch[4 * _NBUF:5 * _NBUF]
    seg0 = wid * _SEG_PER_W

    def _rowcol(s):
        g = seg0 + s
        return (g // _XSPLIT) * _SR, (g % _XSPLIT) * _SW

    def _load_descs(p, s):
        r0, c0 = _rowcol(s)
        sbase = (seg0 + s) * _S
        return (
            (img_hbm.at[pl.ds(r0, _SR), pl.ds(c0, _SW)], bufs[p].at[0]),
            (nz_hbm.at[pl.ds(r0, _SR), pl.ds(c0, _SW)], bufs[p].at[1]),
            (src_hbm.at[pl.ds(sbase, _S)], sbufs[p]),
        )

    def start_loads(p, s):
        for src, dst in _load_descs(p, s):
            pltpu.async_copy(src, dst, lsems[p])

    def wait_loads(p, s):
        for src, dst in _load_descs(p, s):
            pltpu.make_async_copy(src, dst, lsems[p]).wait()

    def _store_desc(p, s):
        r0, c0 = _rowcol(s)
        return obufs[p], out_hbm.at[pl.ds(r0, _SR), pl.ds(c0, _SW)]

    def start_store(p, s):
        src, dst = _store_desc(p, s)
        pltpu.async_copy(src, dst, ssems[p])

    def wait_store(p, s):
        src, dst = _store_desc(p, s)
        pltpu.make_async_copy(src, dst, ssems[p]).wait()

    def merge(p):
        @plsc.parallel_loop(0, _S // _L)
        def grp(t):
            mb = sbufs[p][pl.ds(t * _L, _L)]
            ci = t * _L + iota16
            r = lax.shift_right_logical(ci, (_SW // _L).bit_length() - 1)
            x0 = lax.shift_left(ci & ((_SW // _L) - 1), 4)
            for l in range(_L):
                xl = x0 + l
                tv = plsc.load_gather(bufs[p], [mb, r, xl])
                plsc.store_scatter(obufs[p], [r, xl], tv)

    for p in range(_NBUF):
        start_loads(p, p)

    def rnd(it, carry):
        for p in range(_NBUF):
            s = it * _NBUF + p
            wait_loads(p, s)

            @pl.when(it > 0)
            def _():
                wait_store(p, s - _NBUF)

            merge(p)
            start_store(p, s)

            @pl.when(it < _NROUND - 1)
            def _():
                start_loads(p, s + _NBUF)

        return carry

    lax.fori_loop(0, _NROUND, rnd, 0, unroll=False)
    for p in range(_NBUF):
        wait_store(p, _SEG_PER_W - _NBUF + p)


@jax.jit
def _run(img, nz, srcrel):
    mesh = plsc.VectorSubcoreMesh(
        core_axis_name="c", subcore_axis_name="s",
        num_cores=_NC, num_subcores=_NS,
    )
    f = pl.kernel(
        _sc_body,
        out_type=jax.ShapeDtypeStruct((_ROWS, _W), jnp.float32),
        mesh=mesh,
        scratch_types=(
            [pltpu.VMEM((2, _SR, _SW), jnp.float32)] * _NBUF
            + [pltpu.VMEM((_SR, _SW), jnp.float32)] * _NBUF
            + [pltpu.VMEM((_S,), jnp.int32)] * _NBUF
            + [pltpu.SemaphoreType.DMA] * (2 * _NBUF)
        ),
        compiler_params=pltpu.CompilerParams(needs_layout_passes=False),
    )
    out = f(img.reshape(_ROWS, _W), nz, srcrel)
    return out.reshape(_C, _H, _W)


_NZ, _SRCREL = _consts()


def kernel(img):
    return _run(img, jnp.asarray(_NZ), jnp.asarray(_SRCREL))


# speedup vs baseline: 3.9021x; 1.0062x over previous
import functools

import jax
import jax.numpy as jnp
import numpy as np
from jax import lax
from jax.experimental import pallas as pl
from jax.experimental.pallas import tpu as pltpu
from jax.experimental.pallas import tpu_sc as plsc

_MASKRATIO = 0.75
_PH = 16
_PW = 16
_C = 3
_H = 4096
_W = 4096
_NPH = _H // _PH
_NPW = _W // _PW

_L = 16
_NCHUNK = _C * _H * _W // _L
_ROWS = _C * _H
_SR = 8
_SW = 2048
_S = _SR * _SW // _L
_XSPLIT = _W // _SW
_NSEG = _NCHUNK // _S
_NC = 2
_NS = 16
_NW = _NC * _NS
_SEG_PER_W = _NSEG // _NW
_NBUF = 2
_NROUND = _SEG_PER_W // _NBUF


def _tf2x32(k1, k2, x1, x2):
    R0 = (13, 15, 26, 6)
    R1 = (17, 29, 16, 24)
    k1 = np.uint32(k1)
    k2 = np.uint32(k2)
    ks = (k1, k2, np.uint32(k1 ^ k2 ^ np.uint32(0x1BD11BDA)))
    x = [(x1 + ks[0]).astype(np.uint32), (x2 + ks[1]).astype(np.uint32)]
    for i in range(5):
        for r in R0 if i % 2 == 0 else R1:
            x[0] = (x[0] + x[1]).astype(np.uint32)
            x[1] = ((x[1] << np.uint32(r)) | (x[1] >> np.uint32(32 - r))).astype(
                np.uint32
            )
            x[1] = x[0] ^ x[1]
        x[0] = (x[0] + ks[(i + 1) % 3]).astype(np.uint32)
        x[1] = (x[1] + ks[(i + 2) % 3] + np.uint32(i + 1)).astype(np.uint32)
    return x[0], x[1]


def _random_bits(k1, k2, n):
    lo = np.arange(n, dtype=np.uint32)
    hi = np.zeros(n, dtype=np.uint32)
    b1, b2 = _tf2x32(k1, k2, hi, lo)
    return b1 ^ b2


def _split2(k1, k2):
    b1, b2 = _tf2x32(k1, k2, np.zeros(2, np.uint32), np.arange(2, dtype=np.uint32))
    return (b1[0], b2[0]), (b1[1], b2[1])


def _erfinv_f32(x):
    x = x.astype(np.float32)
    w = (-np.log1p((-x * x).astype(np.float32))).astype(np.float32)
    lt = w < np.float32(5.0)
    wa = (w - np.float32(2.5)).astype(np.float32)
    pa = np.float32(2.81022636e-08)
    for c in (3.43273939e-07, -3.5233877e-06, -4.39150654e-06, 0.00021858087,
              -0.00125372503, -0.00417768164, 0.246640727, 1.50140941):
        pa = (np.float32(c) + pa * wa).astype(np.float32)
    wb = (np.sqrt(np.maximum(w, np.float32(5.0)), dtype=np.float32)
          - np.float32(3.0)).astype(np.float32)
    pb = np.float32(-0.000200214257)
    for c in (0.000100950558, 0.00134934322, -0.00367342844, 0.00573950773,
              -0.0076224613, 0.00943887047, 1.00167406, 2.83297682):
        pb = (np.float32(c) + pb * wb).astype(np.float32)
    return (np.where(lt, pa, pb) * x).astype(np.float32)


def _normal_f32(k1, k2, n):
    bits = _random_bits(k1, k2, n)
    fl = ((bits >> np.uint32(9)) | np.uint32(0x3F800000)).view(np.float32)
    fl = (fl - np.float32(1.0)).astype(np.float32)
    lo = np.nextafter(np.float32(-1.0), np.float32(0.0))
    hi = np.float32(1.0)
    u = np.maximum(lo, (fl * (hi - lo) + lo).astype(np.float32))
    return (np.float32(np.sqrt(2.0)) * _erfinv_f32(u)).astype(np.float32)


@functools.cache
def _consts():
    (m1, m2), (n1, n2) = _split2(np.uint32(0), np.uint32(42))
    (a1, a2), (b1, b2) = _split2(m1, m2)
    higher = _random_bits(a1, a2, _NPH * _NPW)
    lower = _random_bits(b1, b2, _NPH * _NPW)
    span = np.uint32(10)
    mult = np.uint32(((2 ** 16) % 10) ** 2 % 10)
    draws = 1 + (((higher % span) * mult + (lower % span)) % span).astype(np.int32)
    mask = draws < (10.0 * _MASKRATIO)
    noise = _normal_f32(n1, n2, _NPH * _NPW * _PH * _PW * _C).reshape(
        _NPH * _NPW, _PH * _PW * _C
    )
    nz = (
        noise.reshape(_NPH, _NPW, _PH, _PW, _C)
        .transpose(4, 0, 2, 1, 3)
        .reshape(_C, _H, _W)
    )
    g = np.arange(_NSEG, dtype=np.int64)[:, None]
    c = np.arange(_S, dtype=np.int64)[None, :]
    row = (g // _XSPLIT) * _SR + c // (_SW // _L)
    col = (g % _XSPLIT) * _SW + (c % (_SW // _L)) * _L
    y = row % _H
    msel = mask[(y // _PH) * _NPW + (col // _PW)].astype(np.int32)
    return nz, msel.reshape(-1)


def _sc_body(img_hbm, nz_hbm, src_hbm, out_hbm, *scratch):
    wid = lax.axis_index("s") * _NC + lax.axis_index("c")
    iota16 = lax.iota(jnp.int32, _L)
    bufs = scratch[0:_NBUF]
    obufs = scratch[_NBUF:2 * _NBUF]
    sbufs = scratch[2 * _NBUF:3 * _NBUF]
    lsems = scratch[3 * _NBUF:4 * _NBUF]
    ssems = scratch[4 * _NBUF:5 * _NBUF]
    seg0 = wid * _SEG_PER_W

    def _rowcol(s):
        g = seg0 + s
        row = (g // _XSPLIT) * _SR
        return row // _H, row % _H, (g % _XSPLIT) * _SW

    def _load_descs(p, s):
        ch, r0, c0 = _rowcol(s)
        sbase = (seg0 + s) * _S
        return (
            (img_hbm.at[ch, pl.ds(r0, _SR), pl.ds(c0, _SW)], bufs[p].at[0]),
            (nz_hbm.at[ch, pl.ds(r0, _SR), pl.ds(c0, _SW)], bufs[p].at[1]),
            (src_hbm.at[pl.ds(sbase, _S)], sbufs[p]),
        )

    def start_loads(p, s):
        for src, dst in _load_descs(p, s):
            pltpu.async_copy(src, dst, lsems[p])

    def wait_loads(p, s):
        for src, dst in _load_descs(p, s):
            pltpu.make_async_copy(src, dst, lsems[p]).wait()

    def _store_desc(p, s):
        ch, r0, c0 = _rowcol(s)
        return obufs[p], out_hbm.at[ch, pl.ds(r0, _SR), pl.ds(c0, _SW)]

    def start_store(p, s):
        src, dst = _store_desc(p, s)
        pltpu.async_copy(src, dst, ssems[p])

    def wait_store(p, s):
        src, dst = _store_desc(p, s)
        pltpu.make_async_copy(src, dst, ssems[p]).wait()

    def merge(p):
        @plsc.parallel_loop(0, _S // _L)
        def grp(t):
            mb = sbufs[p][pl.ds(t * _L, _L)]
            ci = t * _L + iota16
            r = lax.shift_right_logical(ci, (_SW // _L).bit_length() - 1)
            x0 = lax.shift_left(ci & ((_SW // _L) - 1), 4)
            for l in range(_L):
                xl = x0 + l
                tv = plsc.load_gather(bufs[p], [mb, r, xl])
                plsc.store_scatter(obufs[p], [r, xl], tv)

    for p in range(_NBUF):
        start_loads(p, p)

    def rnd(it, carry):
        for p in range(_NBUF):
            s = it * _NBUF + p
            wait_loads(p, s)

            @pl.when(it > 0)
            def _():
                wait_store(p, s - _NBUF)

            merge(p)
            start_store(p, s)

            @pl.when(it < _NROUND - 1)
            def _():
                start_loads(p, s + _NBUF)

        return carry

    lax.fori_loop(0, _NROUND, rnd, 0, unroll=False)
    for p in range(_NBUF):
        wait_store(p, _SEG_PER_W - _NBUF + p)


@jax.jit
def _run(img, nz, srcrel):
    mesh = plsc.VectorSubcoreMesh(
        core_axis_name="c", subcore_axis_name="s",
        num_cores=_NC, num_subcores=_NS,
    )
    f = pl.kernel(
        _sc_body,
        out_type=jax.ShapeDtypeStruct((_C, _H, _W), jnp.float32),
        mesh=mesh,
        scratch_types=(
            [pltpu.VMEM((2, _SR, _SW), jnp.float32)] * _NBUF
            + [pltpu.VMEM((_SR, _SW), jnp.float32)] * _NBUF
            + [pltpu.VMEM((_S,), jnp.int32)] * _NBUF
            + [pltpu.SemaphoreType.DMA] * (2 * _NBUF)
        ),
        compiler_params=pltpu.CompilerParams(needs_layout_passes=False),
    )
    return f(img, nz, srcrel)


_NZ, _SRCREL = _consts()


def kernel(img):
    return _run(img, jnp.asarray(_NZ), jnp.asarray(_SRCREL))
